# Initial kernel scaffold; baseline (speedup 1.0000x reference)
import jax
import jax.numpy as jnp
from jax.experimental import pallas as pl


def kernel(input, embed):
    raise NotImplementedError("write your pallas kernel here")


# speedup vs baseline: 1.1220x; 1.1220x over previous
import functools

import jax
import jax.numpy as jnp
from jax import lax
from jax.experimental import pallas as pl
from jax.experimental.pallas import tpu as pltpu
from jax.experimental.pallas import tpu_sc as plsc

_DIM = 256
_NE = 8192
_NT = 9216
_TM = 512
_TN = 2048
_NW = 4
_COMMIT = 1.0


def _argmin_body(x_ref, x2_ref, e_ref, e2_ref, ind_ref, loss_ref,
                 r_ref, bidx_ref, bestd_ref):
    m = pl.program_id(0)
    n = pl.program_id(1)
    dot = lax.dot_general(x_ref[...], e_ref[...], (((1,), (0,)), ((), ())),
                          preferred_element_type=jnp.float32)
    dist = (x2_ref[...] - 2.0 * dot) + e2_ref[...]
    wmin = jnp.min(dist, axis=1, keepdims=True)
    iota = lax.broadcasted_iota(jnp.int32, dist.shape, 1)
    warg = jnp.min(jnp.where(dist == wmin, iota, jnp.int32(2**30)),
                   axis=1, keepdims=True) + n * _TN

    @pl.when(n == 0)
    def _():
        r_ref[...] = wmin.astype(jnp.bfloat16).astype(jnp.float32)
        bidx_ref[...] = warg
        bestd_ref[...] = wmin

    @pl.when(n > 0)
    def _():
        upd = wmin < r_ref[...]
        bidx_ref[...] = jnp.where(upd, warg, bidx_ref[...])
        bestd_ref[...] = jnp.where(upd, wmin, bestd_ref[...])
        r_ref[...] = jnp.where(
            upd, wmin.astype(jnp.bfloat16).astype(jnp.float32), r_ref[...])

    @pl.when(n == _NW - 1)
    def _():
        ind_ref[...] = bidx_ref[...]

        @pl.when(m == 0)
        def _():
            loss_ref[...] = jnp.zeros_like(loss_ref)

        loss_ref[...] += jnp.sum(bestd_ref[...], keepdims=True)


def _assign(flatten, x2, embed, e2):
    grid = (_NT // _TM, _NW)
    return pl.pallas_call(
        _argmin_body,
        grid=grid,
        in_specs=[
            pl.BlockSpec((_TM, _DIM), lambda m, n: (m, 0)),
            pl.BlockSpec((_TM, 1), lambda m, n: (m, 0)),
            pl.BlockSpec((_DIM, _TN), lambda m, n: (0, n)),
            pl.BlockSpec((1, _TN), lambda m, n: (0, n)),
        ],
        out_specs=[
            pl.BlockSpec((_TM, 1), lambda m, n: (m, 0)),
            pl.BlockSpec((1, 1), lambda m, n: (0, 0)),
        ],
        out_shape=[
            jax.ShapeDtypeStruct((_NT, 1), jnp.int32),
            jax.ShapeDtypeStruct((1, 1), jnp.float32),
        ],
        scratch_shapes=[
            pltpu.VMEM((_TM, 1), jnp.float32),
            pltpu.VMEM((_TM, 1), jnp.int32),
            pltpu.VMEM((_TM, 1), jnp.float32),
        ],
        name="vq_argmin",
        compiler_params=pltpu.CompilerParams(
            dimension_semantics=("arbitrary", "arbitrary")),
    )(flatten, x2, embed, e2)


_BPW = _NT // 32


def _gather_body(table_hbm, idx_hbm, out_hbm, idx_v, rows_v, sem):
    wid = lax.axis_index("s") * 2 + lax.axis_index("c")
    base = wid * _BPW
    pltpu.sync_copy(idx_hbm.at[pl.ds(base, _BPW)], idx_v)
    pltpu.async_copy(table_hbm.at[idx_v], rows_v, sem).wait()
    pltpu.sync_copy(rows_v, out_hbm.at[pl.ds(base, _BPW)])


def _gather_sc(table, idx):
    mesh = plsc.VectorSubcoreMesh(core_axis_name="c", subcore_axis_name="s")
    k = functools.partial(
        pl.kernel, mesh=mesh,
        out_type=jax.ShapeDtypeStruct((_NT, _DIM), jnp.float32),
        scratch_types=[
            pltpu.VMEM((_BPW,), jnp.int32),
            pltpu.VMEM((_BPW, _DIM), jnp.float32),
            pltpu.SemaphoreType.DMA,
        ],
    )(_gather_body)
    return k(table, idx)


def kernel(input, embed):
    flatten = input.reshape(-1, _DIM)
    x2 = jnp.sum(flatten ** 2, axis=1, keepdims=True)
    e2 = jnp.sum(embed ** 2, axis=0, keepdims=True)
    ind2d, loss = _assign(flatten.astype(jnp.bfloat16), x2,
                          embed.astype(jnp.bfloat16), e2)
    ind = ind2d[:, 0]
    quantize = _gather_sc(embed.T, ind)
    embed_ind = ind.reshape(input.shape[:-1])
    commit_loss = (loss[0, 0] / (_NT * _DIM)) * _COMMIT
    quantize_st = quantize.reshape(input.shape)
    return (quantize_st, embed_ind, commit_loss)


# speedup vs baseline: 1.1872x; 1.0581x over previous
import functools

import jax
import jax.numpy as jnp
from jax import lax
from jax.experimental import pallas as pl
from jax.experimental.pallas import tpu as pltpu
from jax.experimental.pallas import tpu_sc as plsc

_DIM = 256
_NE = 8192
_NT = 9216
_TM = 512
_TN = 2048
_NW = 4
_COMMIT = 1.0


def _argmin_body(x_ref, x2_ref, e_ref, e2_ref, ind_ref, loss_ref,
                 r_ref, bidx_ref, bestd_ref):
    m = pl.program_id(0)
    n = pl.program_id(1)
    dot2 = lax.dot_general(x_ref[...], e_ref[...], (((1,), (0,)), ((), ())),
                           preferred_element_type=jnp.float32)
    dist = (x2_ref[...] + dot2) + e2_ref[...]
    wmin = jnp.min(dist, axis=1, keepdims=True)
    iota = lax.broadcasted_iota(jnp.int32, dist.shape, 1).astype(jnp.float32)
    wargf = jnp.min(jnp.where(dist == wmin, iota, jnp.float32(3e38)),
                    axis=1, keepdims=True)
    warg = wargf.astype(jnp.int32) + n * _TN

    @pl.when(n == 0)
    def _():
        r_ref[...] = wmin.astype(jnp.bfloat16).astype(jnp.float32)
        bidx_ref[...] = warg
        bestd_ref[...] = wmin

    @pl.when(n > 0)
    def _():
        upd = wmin < r_ref[...]
        bidx_ref[...] = jnp.where(upd, warg, bidx_ref[...])
        bestd_ref[...] = jnp.where(upd, wmin, bestd_ref[...])
        r_ref[...] = jnp.where(
            upd, wmin.astype(jnp.bfloat16).astype(jnp.float32), r_ref[...])

    @pl.when(n == _NW - 1)
    def _():
        ind_ref[...] = bidx_ref[...]

        @pl.when(m == 0)
        def _():
            loss_ref[...] = jnp.zeros_like(loss_ref)

        loss_ref[...] += jnp.sum(bestd_ref[...], keepdims=True)


def _assign(flatten, x2, embed, e2):
    grid = (_NT // _TM, _NW)
    return pl.pallas_call(
        _argmin_body,
        grid=grid,
        in_specs=[
            pl.BlockSpec((_TM, _DIM), lambda m, n: (m, 0)),
            pl.BlockSpec((_TM, 1), lambda m, n: (m, 0)),
            pl.BlockSpec((_DIM, _TN), lambda m, n: (0, n)),
            pl.BlockSpec((1, _TN), lambda m, n: (0, n)),
        ],
        out_specs=[
            pl.BlockSpec((_TM, 1), lambda m, n: (m, 0)),
            pl.BlockSpec((1, 1), lambda m, n: (0, 0)),
        ],
        out_shape=[
            jax.ShapeDtypeStruct((_NT, 1), jnp.int32),
            jax.ShapeDtypeStruct((1, 1), jnp.float32),
        ],
        scratch_shapes=[
            pltpu.VMEM((_TM, 1), jnp.float32),
            pltpu.VMEM((_TM, 1), jnp.int32),
            pltpu.VMEM((_TM, 1), jnp.float32),
        ],
        name="vq_argmin",
        compiler_params=pltpu.CompilerParams(
            dimension_semantics=("arbitrary", "arbitrary")),
    )(flatten, x2, embed, e2)


_BPW = _NT // 32


def _gather_body(table_hbm, idx_hbm, out_hbm, idx_v, rows_v, sem):
    wid = lax.axis_index("s") * 2 + lax.axis_index("c")
    base = wid * _BPW
    pltpu.sync_copy(idx_hbm.at[pl.ds(base, _BPW)], idx_v)
    pltpu.async_copy(table_hbm.at[idx_v], rows_v, sem).wait()
    pltpu.sync_copy(rows_v, out_hbm.at[pl.ds(base, _BPW)])


def _gather_sc(table, idx):
    mesh = plsc.VectorSubcoreMesh(core_axis_name="c", subcore_axis_name="s")
    k = functools.partial(
        pl.kernel, mesh=mesh,
        out_type=jax.ShapeDtypeStruct((_NT, _DIM), jnp.float32),
        scratch_types=[
            pltpu.VMEM((_BPW,), jnp.int32),
            pltpu.VMEM((_BPW, _DIM), jnp.float32),
            pltpu.SemaphoreType.DMA,
        ],
    )(_gather_body)
    return k(table, idx)


def kernel(input, embed):
    flatten = input.reshape(-1, _DIM)
    x2 = jnp.sum(flatten ** 2, axis=1, keepdims=True)
    e2 = jnp.sum(embed ** 2, axis=0, keepdims=True)
    ind2d, loss = _assign(flatten.astype(jnp.bfloat16), x2,
                          embed.astype(jnp.bfloat16) * jnp.bfloat16(-2), e2)
    ind = ind2d[:, 0]
    quantize = _gather_sc(embed.T, ind)
    embed_ind = ind.reshape(input.shape[:-1])
    commit_loss = (loss[0, 0] / (_NT * _DIM)) * _COMMIT
    quantize_st = quantize.reshape(input.shape)
    return (quantize_st, embed_ind, commit_loss)


# speedup vs baseline: 1.2593x; 1.0607x over previous
import functools

import jax
import jax.numpy as jnp
from jax import lax
from jax.experimental import pallas as pl
from jax.experimental.pallas import tpu as pltpu
from jax.experimental.pallas import tpu_sc as plsc

_DIM = 256
_NE = 8192
_NT = 9216
_TM = 1024
_TN = 2048
_NW = 4
_COMMIT = 1.0


def _argmin_body(x_ref, x2_ref, e_ref, e2_ref, ind_ref, loss_ref,
                 r_ref, bidx_ref, bestd_ref):
    m = pl.program_id(0)
    n = pl.program_id(1)
    dot2 = lax.dot_general(x_ref[...], e_ref[...], (((1,), (0,)), ((), ())),
                           preferred_element_type=jnp.float32)
    dist = (x2_ref[...] + dot2) + e2_ref[...]
    wmin = jnp.min(dist, axis=1, keepdims=True)
    iota = lax.broadcasted_iota(jnp.int32, dist.shape, 1).astype(jnp.float32)
    wargf = jnp.min(jnp.where(dist == wmin, iota, jnp.float32(3e38)),
                    axis=1, keepdims=True)
    warg = wargf.astype(jnp.int32) + n * _TN

    @pl.when(n == 0)
    def _():
        r_ref[...] = wmin.astype(jnp.bfloat16).astype(jnp.float32)
        bidx_ref[...] = warg
        bestd_ref[...] = wmin

    @pl.when(n > 0)
    def _():
        upd = wmin < r_ref[...]
        bidx_ref[...] = jnp.where(upd, warg, bidx_ref[...])
        bestd_ref[...] = jnp.where(upd, wmin, bestd_ref[...])
        r_ref[...] = jnp.where(
            upd, wmin.astype(jnp.bfloat16).astype(jnp.float32), r_ref[...])

    @pl.when(n == _NW - 1)
    def _():
        ind_ref[...] = bidx_ref[...]

        @pl.when(m == 0)
        def _():
            loss_ref[...] = jnp.zeros_like(loss_ref)

        loss_ref[...] += jnp.sum(bestd_ref[...], keepdims=True)


def _assign(flatten, x2, embed, e2):
    grid = (_NT // _TM, _NW)
    return pl.pallas_call(
        _argmin_body,
        grid=grid,
        in_specs=[
            pl.BlockSpec((_TM, _DIM), lambda m, n: (m, 0)),
            pl.BlockSpec((_TM, 1), lambda m, n: (m, 0)),
            pl.BlockSpec((_DIM, _TN), lambda m, n: (0, n)),
            pl.BlockSpec((1, _TN), lambda m, n: (0, n)),
        ],
        out_specs=[
            pl.BlockSpec((_TM, 1), lambda m, n: (m, 0)),
            pl.BlockSpec((1, 1), lambda m, n: (0, 0)),
        ],
        out_shape=[
            jax.ShapeDtypeStruct((_NT, 1), jnp.int32),
            jax.ShapeDtypeStruct((1, 1), jnp.float32),
        ],
        scratch_shapes=[
            pltpu.VMEM((_TM, 1), jnp.float32),
            pltpu.VMEM((_TM, 1), jnp.int32),
            pltpu.VMEM((_TM, 1), jnp.float32),
        ],
        name="vq_argmin",
        compiler_params=pltpu.CompilerParams(
            dimension_semantics=("arbitrary", "arbitrary")),
    )(flatten, x2, embed, e2)


_BPW = _NT // 32


def _gather_body(table_hbm, idx_hbm, out_hbm, idx_v, rows_v, sem):
    wid = lax.axis_index("s") * 2 + lax.axis_index("c")
    base = wid * _BPW
    pltpu.sync_copy(idx_hbm.at[pl.ds(base, _BPW)], idx_v)
    pltpu.async_copy(table_hbm.at[idx_v], rows_v, sem).wait()
    pltpu.sync_copy(rows_v, out_hbm.at[pl.ds(base, _BPW)])


def _gather_sc(table, idx):
    mesh = plsc.VectorSubcoreMesh(core_axis_name="c", subcore_axis_name="s")
    k = functools.partial(
        pl.kernel, mesh=mesh,
        out_type=jax.ShapeDtypeStruct((_NT, _DIM), jnp.float32),
        scratch_types=[
            pltpu.VMEM((_BPW,), jnp.int32),
            pltpu.VMEM((_BPW, _DIM), jnp.float32),
            pltpu.SemaphoreType.DMA,
        ],
    )(_gather_body)
    return k(table, idx)


def kernel(input, embed):
    flatten = input.reshape(-1, _DIM)
    x2 = jnp.sum(flatten ** 2, axis=1, keepdims=True)
    e2 = jnp.sum(embed ** 2, axis=0, keepdims=True)
    ind2d, loss = _assign(flatten.astype(jnp.bfloat16), x2,
                          embed.astype(jnp.bfloat16) * jnp.bfloat16(-2), e2)
    ind = ind2d[:, 0]
    quantize = _gather_sc(embed.T, ind)
    embed_ind = ind.reshape(input.shape[:-1])
    commit_loss = (loss[0, 0] / (_NT * _DIM)) * _COMMIT
    quantize_st = quantize.reshape(input.shape)
    return (quantize_st, embed_ind, commit_loss)


# speedup vs baseline: 1.2867x; 1.0217x over previous
import functools

import jax
import jax.numpy as jnp
from jax import lax
from jax.experimental import pallas as pl
from jax.experimental.pallas import tpu as pltpu
from jax.experimental.pallas import tpu_sc as plsc

_DIM = 256
_NE = 8192
_NT = 9216
_TM = 1024
_TN = 2048
_NW = 4
_COMMIT = 1.0


def _argmin_body(e_ref, x_ref, e2_ref, x2_ref, ind_ref, loss_ref,
                 r_ref, bidx_ref, bestd_ref):
    m = pl.program_id(0)
    n = pl.program_id(1)
    dot2 = lax.dot_general(e_ref[...], x_ref[...], (((1,), (0,)), ((), ())),
                           preferred_element_type=jnp.float32)
    dist = (x2_ref[...] + dot2) + e2_ref[...]
    d3 = dist.reshape(_TN // 8, 8, _TM)
    cmin = jnp.min(d3, axis=0)
    iota0 = lax.broadcasted_iota(jnp.int32, (_TN // 8, 1, 1), 0
                                 ).astype(jnp.float32)
    carg = jnp.min(jnp.where(d3 == cmin[None], iota0, jnp.float32(3e38)),
                   axis=0)
    siota = lax.broadcasted_iota(jnp.int32, (8, _TM), 0).astype(jnp.float32)
    j8 = carg * 8.0 + siota
    wmin = jnp.min(cmin, axis=0, keepdims=True)
    wargf = jnp.min(jnp.where(cmin == wmin, j8, jnp.float32(3e38)),
                    axis=0, keepdims=True)
    warg = wargf.astype(jnp.int32) + n * _TN

    @pl.when(n == 0)
    def _():
        r_ref[...] = wmin.astype(jnp.bfloat16).astype(jnp.float32)
        bidx_ref[...] = warg
        bestd_ref[...] = wmin

    @pl.when(n > 0)
    def _():
        upd = wmin < r_ref[...]
        bidx_ref[...] = jnp.where(upd, warg, bidx_ref[...])
        bestd_ref[...] = jnp.where(upd, wmin, bestd_ref[...])
        r_ref[...] = jnp.where(
            upd, wmin.astype(jnp.bfloat16).astype(jnp.float32), r_ref[...])

    @pl.when(n == _NW - 1)
    def _():
        ind_ref[...] = bidx_ref[...].reshape(1, 1, _TM)

        @pl.when(m == 0)
        def _():
            loss_ref[...] = jnp.zeros_like(loss_ref)

        loss_ref[...] += jnp.sum(bestd_ref[...], keepdims=True)


def _assign(eT, xT, e2T, x2T):
    grid = (_NT // _TM, _NW)
    return pl.pallas_call(
        _argmin_body,
        grid=grid,
        in_specs=[
            pl.BlockSpec((_TN, _DIM), lambda m, n: (n, 0)),
            pl.BlockSpec((_DIM, _TM), lambda m, n: (0, m)),
            pl.BlockSpec((_TN, 1), lambda m, n: (n, 0)),
            pl.BlockSpec((1, _TM), lambda m, n: (0, m)),
        ],
        out_specs=[
            pl.BlockSpec((1, 1, _TM), lambda m, n: (m, 0, 0)),
            pl.BlockSpec((1, 1), lambda m, n: (0, 0)),
        ],
        out_shape=[
            jax.ShapeDtypeStruct((_NT // _TM, 1, _TM), jnp.int32),
            jax.ShapeDtypeStruct((1, 1), jnp.float32),
        ],
        scratch_shapes=[
            pltpu.VMEM((1, _TM), jnp.float32),
            pltpu.VMEM((1, _TM), jnp.int32),
            pltpu.VMEM((1, _TM), jnp.float32),
        ],
        name="vq_argmin",
        compiler_params=pltpu.CompilerParams(
            dimension_semantics=("arbitrary", "arbitrary")),
    )(eT, xT, e2T, x2T)


_BPW = _NT // 32


def _gather_body(table_hbm, idx_hbm, out_hbm, idx_v, rows_v, sem):
    wid = lax.axis_index("s") * 2 + lax.axis_index("c")
    base = wid * _BPW
    pltpu.sync_copy(idx_hbm.at[pl.ds(base, _BPW)], idx_v)
    pltpu.async_copy(table_hbm.at[idx_v], rows_v, sem).wait()
    pltpu.sync_copy(rows_v, out_hbm.at[pl.ds(base, _BPW)])


def _gather_sc(table, idx):
    mesh = plsc.VectorSubcoreMesh(core_axis_name="c", subcore_axis_name="s")
    k = functools.partial(
        pl.kernel, mesh=mesh,
        out_type=jax.ShapeDtypeStruct((_NT, _DIM), jnp.float32),
        scratch_types=[
            pltpu.VMEM((_BPW,), jnp.int32),
            pltpu.VMEM((_BPW, _DIM), jnp.float32),
            pltpu.SemaphoreType.DMA,
        ],
    )(_gather_body)
    return k(table, idx)


def kernel(input, embed):
    flatten = input.reshape(-1, _DIM)
    x2 = jnp.sum(flatten ** 2, axis=1, keepdims=True)
    e2 = jnp.sum(embed ** 2, axis=0, keepdims=True)
    eT = jnp.transpose(embed.astype(jnp.bfloat16) * jnp.bfloat16(-2))
    xT = jnp.transpose(flatten.astype(jnp.bfloat16))
    ind3d, loss = _assign(eT, xT, jnp.transpose(e2), jnp.transpose(x2))
    ind = ind3d.reshape(-1)
    quantize = _gather_sc(embed.T, ind)
    embed_ind = ind.reshape(input.shape[:-1])
    commit_loss = (loss[0, 0] / (_NT * _DIM)) * _COMMIT
    quantize_st = quantize.reshape(input.shape)
    return (quantize_st, embed_ind, commit_loss)
